# baseline (speedup 1.0000x reference)
import functools

import jax
import jax.numpy as jnp
from jax import lax
from jax.experimental import pallas as pl
from jax.experimental.pallas import tpu as pltpu
from jax.experimental.pallas import tpu_sc as plsc


_CHUNK_ROWS = 16


def kernel(x):
    b, s, d = x.shape
    rows = b * s
    x2 = x.reshape(rows, d)
    info = plsc.get_sparse_core_info()
    nc, ns = info.num_cores, info.num_subcores
    nw = nc * ns
    rpw = rows // nw
    n_chunks = rpw // _CHUNK_ROWS
    mesh = plsc.VectorSubcoreMesh(core_axis_name="c", subcore_axis_name="s")

    @functools.partial(
        pl.kernel,
        out_type=jax.ShapeDtypeStruct((rows, d), x.dtype),
        mesh=mesh,
        scratch_types=[
            pltpu.VMEM((_CHUNK_ROWS, d), x.dtype),
            pltpu.VMEM((_CHUNK_ROWS, d), x.dtype),
            pltpu.VMEM((_CHUNK_ROWS, d), x.dtype),
            pltpu.SemaphoreType.DMA,
            pltpu.SemaphoreType.DMA,
            pltpu.SemaphoreType.DMA,
            pltpu.SemaphoreType.DMA,
            pltpu.SemaphoreType.DMA,
            pltpu.SemaphoreType.DMA,
        ],
    )
    def sc_copy(x_hbm, o_hbm, buf0, buf1, buf2,
                rsem0, rsem1, rsem2, wsem0, wsem1, wsem2):
        wid = lax.axis_index("s") * nc + lax.axis_index("c")
        base = wid * rpw
        bufs = (buf0, buf1, buf2)
        rsems = (rsem0, rsem1, rsem2)
        wsems = (wsem0, wsem1, wsem2)

        def rd(j):
            return pltpu.async_copy(
                x_hbm.at[pl.ds(base + j * _CHUNK_ROWS, _CHUNK_ROWS), :],
                bufs[j % 3],
                rsems[j % 3],
            )

        def wr(j):
            return pltpu.async_copy(
                bufs[j % 3],
                o_hbm.at[pl.ds(base + j * _CHUNK_ROWS, _CHUNK_ROWS), :],
                wsems[j % 3],
            )

        reads = {0: rd(0)}
        writes = {}
        for j in range(n_chunks):
            reads.pop(j).wait()
            writes[j] = wr(j)
            if j + 1 < n_chunks:
                if j >= 2:
                    writes.pop(j - 2).wait()
                reads[j + 1] = rd(j + 1)
        for h in writes.values():
            h.wait()

    return sc_copy(x2).reshape(b, s, d)


# speedup vs baseline: 1.3949x; 1.3949x over previous
import jax
import jax.numpy as jnp
from jax.experimental import pallas as pl
from jax.experimental.pallas import tpu as pltpu


_BLOCK_ROWS = 1024


def _copy_body(x_ref, o_ref):
    o_ref[...] = x_ref[...]


def kernel(x):
    b, s, d = x.shape
    rows = b * s
    x2 = x.reshape(rows, d)
    out = pl.pallas_call(
        _copy_body,
        grid=(rows // _BLOCK_ROWS,),
        in_specs=[pl.BlockSpec((_BLOCK_ROWS, d), lambda i: (i, 0))],
        out_specs=pl.BlockSpec((_BLOCK_ROWS, d), lambda i: (i, 0)),
        out_shape=jax.ShapeDtypeStruct((rows, d), x.dtype),
        compiler_params=pltpu.CompilerParams(
            dimension_semantics=("parallel",),
        ),
    )(x2)
    return out.reshape(b, s, d)


# speedup vs baseline: 1.3951x; 1.0002x over previous
import jax
from jax.experimental import pallas as pl
from jax.experimental.pallas import tpu as pltpu


_BLOCK_ROWS = 1024


def _copy_body(x_ref, o_ref):
    o_ref[...] = x_ref[...]


def kernel(x):
    b, s, d = x.shape
    rows = b * s
    x2 = x.reshape(rows, d)
    out = pl.pallas_call(
        _copy_body,
        grid=(rows // _BLOCK_ROWS,),
        in_specs=[pl.BlockSpec((_BLOCK_ROWS, d), lambda i: (i, 0))],
        out_specs=pl.BlockSpec((_BLOCK_ROWS, d), lambda i: (i, 0)),
        out_shape=jax.ShapeDtypeStruct((rows, d), x.dtype),
        compiler_params=pltpu.CompilerParams(
            dimension_semantics=("parallel",),
        ),
    )(x2)
    return out.reshape(b, s, d)
